# baseline (speedup 1.0000x reference)
import functools

import jax
import jax.numpy as jnp
from jax import lax
from jax.experimental import pallas as pl
from jax.experimental.pallas import tpu as pltpu
from jax.experimental.pallas import tpu_sc as plsc

_L = 16
_NC = 2
_NS = 16
_NW = _NC * _NS
_CHUNK = 128


def _mm2_body(x_ref, wl_ref, bl_ref, wr_ref, br_ref, xl_ref, xr_ref):
    x = x_ref[...]
    xl_ref[...] = jnp.dot(x, wl_ref[...], preferred_element_type=jnp.float32) + bl_ref[...]
    xr_ref[...] = jnp.dot(x, wr_ref[...], preferred_element_type=jnp.float32) + br_ref[...]


def _mm2(x, Wl, bl, Wr, br):
    N, Din = x.shape
    H = Wl.shape[1]
    blk = 400
    return pl.pallas_call(
        _mm2_body,
        grid=(N // blk,),
        in_specs=[
            pl.BlockSpec((blk, Din), lambda i: (i, 0)),
            pl.BlockSpec((Din, H), lambda i: (0, 0)),
            pl.BlockSpec((1, H), lambda i: (0, 0)),
            pl.BlockSpec((Din, H), lambda i: (0, 0)),
            pl.BlockSpec((1, H), lambda i: (0, 0)),
        ],
        out_specs=[
            pl.BlockSpec((blk, H), lambda i: (i, 0)),
            pl.BlockSpec((blk, H), lambda i: (i, 0)),
        ],
        out_shape=[
            jax.ShapeDtypeStruct((N, H), jnp.float32),
            jax.ShapeDtypeStruct((N, H), jnp.float32),
        ],
    )(x, Wl, bl.reshape(1, H), Wr, br.reshape(1, H))


def _norm_mm2_body(acc_ref, den_ref, b_ref, wl_ref, bl_ref, wr_ref, br_ref,
                   xl_ref, xr_ref):
    acc = acc_ref[0] + acc_ref[1]
    den = den_ref[:, 0:1] + den_ref[:, 1:2]
    h = jnp.maximum(acc / (den + 1e-16) + b_ref[...], 0.0)
    xl_ref[...] = jnp.dot(h, wl_ref[...], preferred_element_type=jnp.float32) + bl_ref[...]
    xr_ref[...] = jnp.dot(h, wr_ref[...], preferred_element_type=jnp.float32) + br_ref[...]


def _norm_mm2(acc_parts, denT, bias, Wl, bl, Wr, br):
    _, N, C = acc_parts.shape
    H = Wl.shape[1]
    blk = 400
    return pl.pallas_call(
        _norm_mm2_body,
        grid=(N // blk,),
        in_specs=[
            pl.BlockSpec((2, blk, C), lambda i: (0, i, 0)),
            pl.BlockSpec((blk, 2), lambda i: (i, 0)),
            pl.BlockSpec((1, C), lambda i: (0, 0)),
            pl.BlockSpec((C, H), lambda i: (0, 0)),
            pl.BlockSpec((1, H), lambda i: (0, 0)),
            pl.BlockSpec((C, H), lambda i: (0, 0)),
            pl.BlockSpec((1, H), lambda i: (0, 0)),
        ],
        out_specs=[
            pl.BlockSpec((blk, H), lambda i: (i, 0)),
            pl.BlockSpec((blk, H), lambda i: (i, 0)),
        ],
        out_shape=[
            jax.ShapeDtypeStruct((N, H), jnp.float32),
            jax.ShapeDtypeStruct((N, H), jnp.float32),
        ],
    )(acc_parts, denT, bias.reshape(1, C), Wl, bl.reshape(1, H), Wr, br.reshape(1, H))


def _norm_final_body(acc_ref, den_ref, b_ref, wpv_ref, h_ref, g_ref):
    acc = acc_ref[0] + acc_ref[1]
    den = den_ref[:, 0:1] + den_ref[:, 1:2]
    h = jnp.maximum(acc / (den + 1e-16) + b_ref[...], 0.0)
    h_ref[...] = h
    g_ref[...] = h * wpv_ref[...]


def _norm_final(acc_parts, denT, bias, wpv):
    _, N, C = acc_parts.shape
    blk = 400
    return pl.pallas_call(
        _norm_final_body,
        grid=(N // blk,),
        in_specs=[
            pl.BlockSpec((2, blk, C), lambda i: (0, i, 0)),
            pl.BlockSpec((blk, 2), lambda i: (i, 0)),
            pl.BlockSpec((1, C), lambda i: (0, 0)),
            pl.BlockSpec((1, C), lambda i: (0, 0)),
        ],
        out_specs=[
            pl.BlockSpec((blk, C), lambda i: (i, 0)),
            pl.BlockSpec((blk, C), lambda i: (i, 0)),
        ],
        out_shape=[
            jax.ShapeDtypeStruct((N, C), jnp.float32),
            jax.ShapeDtypeStruct((N, C), jnp.float32),
        ],
    )(acc_parts, denT, bias.reshape(1, C), wpv.reshape(1, C))


def _edge_sc(C, N, E_tot, E_pad, CHUNK):
    per_w = E_pad // _NW
    n_chunks = per_w // CHUNK
    assert n_chunks % 2 == 1 and n_chunks >= 3
    cb_n = C // _L
    rows_per_tile = (N // _NS) // 8 * 8
    rows_tail = N - rows_per_tile * _NS

    mesh = plsc.VectorSubcoreMesh(core_axis_name="c", subcore_axis_name="s")

    @functools.partial(
        pl.kernel,
        out_type=[
            jax.ShapeDtypeStruct((_NC, N, C), jnp.float32),
            jax.ShapeDtypeStruct((_NC, 8, N), jnp.float32),
        ],
        mesh=mesh,
        scratch_types=[
            [pltpu.VMEM((CHUNK,), jnp.int32)] * 2,
            [pltpu.VMEM((CHUNK,), jnp.int32)] * 2,
            [pltpu.VMEM((CHUNK,), jnp.int32)] * 2,
            [pltpu.VMEM((CHUNK, C), jnp.float32)] * 2,
            [pltpu.VMEM((CHUNK, C), jnp.float32)] * 2,
            [pltpu.VMEM((CHUNK,), jnp.float32)] * 2,
            pltpu.VMEM((C,), jnp.float32),
            pltpu.VMEM((_L, 32), jnp.float32),
            pltpu.VMEM_SHARED((N, C), jnp.float32),
            pltpu.VMEM_SHARED((N,), jnp.float32),
            [pltpu.SemaphoreType.DMA] * 2,
            [pltpu.SemaphoreType.DMA] * 2,
            [pltpu.SemaphoreType.DMA] * 2,
        ],
        compiler_params=pltpu.CompilerParams(use_tc_tiling_on_sc=False),
    )
    def k(xl_hbm, xr_hbm, src_hbm, dst_hbm, att_hbm,
          acc_out, den_out,
          sidx2, didx2, didxs2, srows2, drows2, exbuf2, attv, tb, acc_sh,
          den_sh, isem, gsem, ssem):
        cid = lax.axis_index("c")
        sid = lax.axis_index("s")
        w = cid * _NS + sid

        pltpu.sync_copy(att_hbm, attv)

        zbuf = drows2[0]

        def _zero_row(i, _):
            zbuf[i, :] = jnp.zeros((C,), jnp.float32)
            return 0
        lax.fori_loop(0, CHUNK, _zero_row, 0)
        row0 = sid * rows_per_tile
        nfull = rows_per_tile // CHUNK
        rem = rows_per_tile - nfull * CHUNK
        for i in range(nfull):
            pltpu.sync_copy(zbuf, acc_sh.at[pl.ds(row0 + i * CHUNK, CHUNK)])
        if rem:
            pltpu.sync_copy(zbuf.at[pl.ds(0, rem)],
                            acc_sh.at[pl.ds(row0 + nfull * CHUNK, rem)])

        @pl.when(sid == _NS - 1)
        def _():
            pltpu.sync_copy(zbuf.at[pl.ds(0, rows_tail)],
                            acc_sh.at[pl.ds(rows_per_tile * _NS, rows_tail)])

        zex = exbuf2[0]
        zex[...] = jnp.zeros((CHUNK,), jnp.float32)

        @pl.when(sid == 0)
        def _():
            for i in range(N // CHUNK):
                pltpu.sync_copy(zex, den_sh.at[pl.ds(i * CHUNK, CHUNK)])
            dr = N - (N // CHUNK) * CHUNK
            if dr:
                pltpu.sync_copy(zex.at[pl.ds(0, dr)],
                                den_sh.at[pl.ds((N // CHUNK) * CHUNK, dr)])

        plsc.subcore_barrier()

        att_regs = [attv[pl.ds(cb * _L, _L)] for cb in range(cb_n)]
        fiota = lax.iota(jnp.int32, _L).astype(jnp.float32)

        def idx_src(t):
            base = w * per_w + t * CHUNK
            return (src_hbm.at[pl.ds(base, CHUNK)], dst_hbm.at[pl.ds(base, CHUNK)])

        def issue_idx(t, p):
            s_src, d_src = idx_src(t)
            pltpu.async_copy(s_src, sidx2[p], isem[p])
            pltpu.async_copy(d_src, didx2[p], isem[p])

        def wait_idx(t, p):
            s_src, d_src = idx_src(t)
            pltpu.make_async_copy(s_src, sidx2[p], isem[p]).wait()
            pltpu.make_async_copy(d_src, didx2[p], isem[p]).wait()

        def issue_gather(p):
            pltpu.async_copy(xl_hbm.at[sidx2[p]], srows2[p], gsem[p])
            pltpu.async_copy(xr_hbm.at[didx2[p]], drows2[p], gsem[p])

        def wait_gather(p):
            pltpu.make_async_copy(xl_hbm.at[sidx2[p]], srows2[p], gsem[p]).wait()
            pltpu.make_async_copy(xr_hbm.at[didx2[p]], drows2[p], gsem[p]).wait()

        def issue_scatter(p):
            pltpu.async_copy(drows2[p], acc_sh.at[didxs2[p]], ssem[p], add=True)
            pltpu.async_copy(exbuf2[p], den_sh.at[didxs2[p]], ssem[p], add=True)

        def wait_scatter(p):
            pltpu.make_async_copy(drows2[p], acc_sh.at[didxs2[p]], ssem[p]).wait()
            pltpu.make_async_copy(exbuf2[p], den_sh.at[didxs2[p]], ssem[p]).wait()

        def snap_didx(p):
            for q in range(CHUNK // _L):
                didxs2[p][pl.ds(q * _L, _L)] = didx2[p][pl.ds(q * _L, _L)]

        def compute(t, p):
            srows, drows, exbuf = srows2[p], drows2[p], exbuf2[p]
            base = w * per_w + t * CHUNK
            def group_body(g, _):
                e0 = g * _L
                ex_acc = jnp.zeros((_L,), jnp.float32)
                for jj in range(_L):
                    e = e0 + jj
                    acc = jnp.zeros((_L,), jnp.float32)
                    svals = []
                    for cb in range(cb_n):
                        sv = srows[e, pl.ds(cb * _L, _L)]
                        svals.append(sv)
                        u = sv + drows[e, pl.ds(cb * _L, _L)]
                        lr = jnp.maximum(u, 0.2 * u)
                        acc = acc + lr * att_regs[cb]
                    tb[jj, pl.ds(0, _L)] = acc
                    acc = acc + tb[jj, pl.ds(8, _L)]
                    parts = [acc[c] for c in range(8)]
                    while len(parts) > 1:
                        parts = [a + b for a, b in zip(parts[::2], parts[1::2])]
                    logit = jnp.where((base + e) < E_tot, parts[0], -1e30)
                    exv = jnp.exp(jnp.full((_L,), logit, jnp.float32))
                    for cb in range(cb_n):
                        drows[e, pl.ds(cb * _L, _L)] = svals[cb] * exv
                    onehot = jnp.maximum(
                        1.0 - jnp.abs(fiota - float(jj)), 0.0)
                    ex_acc = ex_acc + exv * onehot
                exbuf[pl.ds(e0, _L)] = ex_acc
                return 0
            lax.fori_loop(0, CHUNK // _L, group_body, 0)

        issue_idx(0, 0)
        issue_idx(1, 1)
        wait_idx(0, 0)
        wait_idx(1, 1)
        issue_gather(0)
        issue_gather(1)
        wait_gather(0)
        snap_didx(0)
        issue_idx(2, 0)
        compute(0, 0)
        issue_scatter(0)

        def loop_body(i, _):
            for b in range(2):
                t = 1 + 2 * i + b
                p = (1 + b) % 2

                @pl.when(t < n_chunks - 1)
                def _():
                    wait_idx(t + 1, 1 - p)
                wait_scatter(1 - p)

                @pl.when(t < n_chunks - 1)
                def _():
                    issue_gather(1 - p)
                wait_gather(p)
                snap_didx(p)

                @pl.when(t < n_chunks - 2)
                def _():
                    issue_idx(t + 2, p)
                compute(t, p)
                issue_scatter(p)
            return 0

        lax.fori_loop(0, (n_chunks - 1) // 2, loop_body, 0)
        wait_scatter(0)

        plsc.subcore_barrier()

        pltpu.sync_copy(acc_sh.at[pl.ds(row0, rows_per_tile)],
                        acc_out.at[cid, pl.ds(row0, rows_per_tile)])

        @pl.when(sid == _NS - 1)
        def _():
            pltpu.sync_copy(acc_sh.at[pl.ds(rows_per_tile * _NS, rows_tail)],
                            acc_out.at[cid, pl.ds(rows_per_tile * _NS, rows_tail)])

        @pl.when(sid == 0)
        def _():
            pltpu.sync_copy(den_sh, den_out.at[cid, 0])

    return k


def _decode_sc(C, N, E_tot, E_pad):
    per_w = E_pad // _NW
    n_chunks = per_w // _CHUNK
    assert n_chunks % 2 == 1 and n_chunks >= 3
    cb_n = C // _L

    mesh = plsc.VectorSubcoreMesh(core_axis_name="c", subcore_axis_name="s")

    @functools.partial(
        pl.kernel,
        out_type=jax.ShapeDtypeStruct((E_pad,), jnp.float32),
        mesh=mesh,
        scratch_types=[
            pltpu.VMEM((n_chunks, _CHUNK), jnp.int32),
            pltpu.VMEM((n_chunks, _CHUNK), jnp.int32),
            [pltpu.VMEM((_CHUNK, C), jnp.float32)] * 2,
            [pltpu.VMEM((_CHUNK, C), jnp.float32)] * 2,
            [pltpu.VMEM((_CHUNK,), jnp.float32)] * 2,
            pltpu.VMEM((_L,), jnp.float32),
            pltpu.VMEM((_L, 32), jnp.float32),
            [pltpu.SemaphoreType.DMA] * 2,
            [pltpu.SemaphoreType.DMA] * 2,
        ],
        compiler_params=pltpu.CompilerParams(use_tc_tiling_on_sc=False),
    )
    def k(g_hbm, h_hbm, src_hbm, dst_hbm, bps_hbm, res_out,
          sidx2, didx2, grows2, hrows2, resbuf2, bpsv, tb, gsem, rsem):
        cid = lax.axis_index("c")
        sid = lax.axis_index("s")
        w = cid * _NS + sid

        pltpu.sync_copy(bps_hbm, bpsv)
        bps = bpsv[...]
        fiota = lax.iota(jnp.int32, _L).astype(jnp.float32)

        pltpu.sync_copy(src_hbm.at[pl.ds(w * n_chunks, n_chunks)], sidx2)
        pltpu.sync_copy(dst_hbm.at[pl.ds(w * n_chunks, n_chunks)], didx2)

        def issue_gather(t, p):
            pltpu.async_copy(g_hbm.at[sidx2.at[t]], grows2[p], gsem[p])
            pltpu.async_copy(h_hbm.at[didx2.at[t]], hrows2[p], gsem[p])

        def wait_gather(t, p):
            pltpu.make_async_copy(g_hbm.at[sidx2.at[t]], grows2[p], gsem[p]).wait()
            pltpu.make_async_copy(h_hbm.at[didx2.at[t]], hrows2[p], gsem[p]).wait()

        def out_ref(t):
            base = w * per_w + t * _CHUNK
            return res_out.at[pl.ds(pl.multiple_of(base, _CHUNK), _CHUNK)]

        def issue_store(t, p):
            pltpu.async_copy(resbuf2[p], out_ref(t), rsem[p])

        def wait_store(t, p):
            pltpu.make_async_copy(resbuf2[p], out_ref(t), rsem[p]).wait()

        def compute(t, p):
            grows, hrows, resbuf = grows2[p], hrows2[p], resbuf2[p]

            def group_body(g, _):
                e0 = g * _L
                r_acc = bps
                for jj in range(_L):
                    e = e0 + jj
                    acc = jnp.zeros((_L,), jnp.float32)
                    for cb in range(cb_n):
                        acc = acc + (grows[e, pl.ds(cb * _L, _L)]
                                     * hrows[e, pl.ds(cb * _L, _L)])
                    tb[jj, pl.ds(0, _L)] = acc
                    acc = acc + tb[jj, pl.ds(8, _L)]
                    parts = [acc[c] for c in range(8)]
                    while len(parts) > 1:
                        parts = [a + b for a, b in zip(parts[::2], parts[1::2])]
                    rv = jnp.full((_L,), parts[0], jnp.float32)
                    onehot = jnp.maximum(
                        1.0 - jnp.abs(fiota - float(jj)), 0.0)
                    r_acc = r_acc + rv * onehot
                resbuf[pl.ds(e0, _L)] = r_acc
                return 0
            lax.fori_loop(0, _CHUNK // _L, group_body, 0)

        issue_gather(0, 0)
        issue_gather(1, 1)
        wait_gather(0, 0)
        compute(0, 0)
        issue_store(0, 0)

        def loop_body(i, _):
            for b in range(2):
                t = 1 + 2 * i + b
                p = (1 + b) % 2

                @pl.when(t + 1 < n_chunks)
                def _():
                    issue_gather(t + 1, 1 - p)
                wait_gather(t, p)
                compute(t, p)
                wait_store(t - 1, 1 - p)
                issue_store(t, p)
            return 0

        lax.fori_loop(0, (n_chunks - 1) // 2, loop_body, 0)
        wait_store(n_chunks - 1, 0)

    return k


def _pad1d(a, n):
    return jnp.concatenate([a, jnp.zeros((n - a.shape[0],), a.dtype)])


def _padded_len(e, gran):
    k = (e + gran - 1) // gran
    if k % 2 == 0:
        k += 1
    return k * gran


def kernel(x, edge_index, edge_label_index, Wl1, bl1, Wr1, br1, att1, bias1,
           Wl2, bl2, Wr2, br2, att2, bias2, Wp, bp):
    N = x.shape[0]
    E = edge_index.shape[1]
    E_tot = E + N
    E_pad1 = _padded_len(E_tot, _NW * 64)
    E_pad2 = _padded_len(E_tot, _NW * 128)
    EL = edge_label_index.shape[1]
    EL_pad = _padded_len(EL, _NW * _CHUNK)

    loop = jnp.arange(N, dtype=edge_index.dtype)
    src_raw = jnp.concatenate([edge_index[0], loop])
    dst_raw = jnp.concatenate([edge_index[1], loop])
    src1, dst1 = _pad1d(src_raw, E_pad1), _pad1d(dst_raw, E_pad1)
    src2, dst2 = _pad1d(src_raw, E_pad2), _pad1d(dst_raw, E_pad2)

    xl1, xr1 = _mm2(x, Wl1, bl1, Wr1, br1)
    acc1, den1 = _edge_sc(128, N, E_tot, E_pad1, 64)(xl1, xr1, src1, dst1, att1)
    xl2, xr2 = _norm_mm2(acc1, den1[:, 0, :].T, bias1, Wl2, bl2, Wr2, br2)

    acc2, den2 = _edge_sc(64, N, E_tot, E_pad2, 128)(xl2, xr2, src2, dst2, att2)
    wpv = Wp @ jnp.ones((2,), jnp.float32)
    h2, g2 = _norm_final(acc2, den2[:, 0, :].T, bias2, wpv)

    srcL = _pad1d(edge_label_index[0], EL_pad).reshape(-1, _CHUNK)
    dstL = _pad1d(edge_label_index[1], EL_pad).reshape(-1, _CHUNK)
    bps = jnp.full((_L,), jnp.sum(bp), jnp.float32)
    res = _decode_sc(64, N, EL, EL_pad)(g2, h2, srcL, dstL, bps)
    return res[:EL]


# speedup vs baseline: 1.0010x; 1.0010x over previous
import functools

import jax
import jax.numpy as jnp
from jax import lax
from jax.experimental import pallas as pl
from jax.experimental.pallas import tpu as pltpu
from jax.experimental.pallas import tpu_sc as plsc

_L = 16
_NC = 2
_NS = 16
_NW = _NC * _NS
_CHUNK = 128


def _mm2_body(x_ref, wl_ref, bl_ref, wr_ref, br_ref, xl_ref, xr_ref):
    x = x_ref[...]
    xl_ref[...] = jnp.dot(x, wl_ref[...], preferred_element_type=jnp.float32) + bl_ref[...]
    xr_ref[...] = jnp.dot(x, wr_ref[...], preferred_element_type=jnp.float32) + br_ref[...]


def _mm2(x, Wl, bl, Wr, br):
    N, Din = x.shape
    H = Wl.shape[1]
    blk = 400
    return pl.pallas_call(
        _mm2_body,
        grid=(N // blk,),
        in_specs=[
            pl.BlockSpec((blk, Din), lambda i: (i, 0)),
            pl.BlockSpec((Din, H), lambda i: (0, 0)),
            pl.BlockSpec((1, H), lambda i: (0, 0)),
            pl.BlockSpec((Din, H), lambda i: (0, 0)),
            pl.BlockSpec((1, H), lambda i: (0, 0)),
        ],
        out_specs=[
            pl.BlockSpec((blk, H), lambda i: (i, 0)),
            pl.BlockSpec((blk, H), lambda i: (i, 0)),
        ],
        out_shape=[
            jax.ShapeDtypeStruct((N, H), jnp.float32),
            jax.ShapeDtypeStruct((N, H), jnp.float32),
        ],
    )(x, Wl, bl.reshape(1, H), Wr, br.reshape(1, H))


def _norm_mm2_body(acc_ref, den_ref, b_ref, wl_ref, bl_ref, wr_ref, br_ref,
                   xl_ref, xr_ref):
    acc = acc_ref[0] + acc_ref[1]
    den = den_ref[:, 0:1] + den_ref[:, 1:2]
    h = jnp.maximum(acc / (den + 1e-16) + b_ref[...], 0.0)
    xl_ref[...] = jnp.dot(h, wl_ref[...], preferred_element_type=jnp.float32) + bl_ref[...]
    xr_ref[...] = jnp.dot(h, wr_ref[...], preferred_element_type=jnp.float32) + br_ref[...]


def _norm_mm2(acc_parts, denT, bias, Wl, bl, Wr, br):
    _, N, C = acc_parts.shape
    H = Wl.shape[1]
    blk = 400
    return pl.pallas_call(
        _norm_mm2_body,
        grid=(N // blk,),
        in_specs=[
            pl.BlockSpec((2, blk, C), lambda i: (0, i, 0)),
            pl.BlockSpec((blk, 2), lambda i: (i, 0)),
            pl.BlockSpec((1, C), lambda i: (0, 0)),
            pl.BlockSpec((C, H), lambda i: (0, 0)),
            pl.BlockSpec((1, H), lambda i: (0, 0)),
            pl.BlockSpec((C, H), lambda i: (0, 0)),
            pl.BlockSpec((1, H), lambda i: (0, 0)),
        ],
        out_specs=[
            pl.BlockSpec((blk, H), lambda i: (i, 0)),
            pl.BlockSpec((blk, H), lambda i: (i, 0)),
        ],
        out_shape=[
            jax.ShapeDtypeStruct((N, H), jnp.float32),
            jax.ShapeDtypeStruct((N, H), jnp.float32),
        ],
    )(acc_parts, denT, bias.reshape(1, C), Wl, bl.reshape(1, H), Wr, br.reshape(1, H))


def _norm_final_body(acc_ref, den_ref, b_ref, wpv_ref, h_ref, g_ref):
    acc = acc_ref[0] + acc_ref[1]
    den = den_ref[:, 0:1] + den_ref[:, 1:2]
    h = jnp.maximum(acc / (den + 1e-16) + b_ref[...], 0.0)
    h_ref[...] = h
    g_ref[...] = h * wpv_ref[...]


def _norm_final(acc_parts, denT, bias, wpv):
    _, N, C = acc_parts.shape
    blk = 400
    return pl.pallas_call(
        _norm_final_body,
        grid=(N // blk,),
        in_specs=[
            pl.BlockSpec((2, blk, C), lambda i: (0, i, 0)),
            pl.BlockSpec((blk, 2), lambda i: (i, 0)),
            pl.BlockSpec((1, C), lambda i: (0, 0)),
            pl.BlockSpec((1, C), lambda i: (0, 0)),
        ],
        out_specs=[
            pl.BlockSpec((blk, C), lambda i: (i, 0)),
            pl.BlockSpec((blk, C), lambda i: (i, 0)),
        ],
        out_shape=[
            jax.ShapeDtypeStruct((N, C), jnp.float32),
            jax.ShapeDtypeStruct((N, C), jnp.float32),
        ],
    )(acc_parts, denT, bias.reshape(1, C), wpv.reshape(1, C))


def _edge_sc(C, N, E_tot, E_pad, CHUNK):
    per_w = E_pad // _NW
    n_chunks = per_w // CHUNK
    assert n_chunks % 2 == 1 and n_chunks >= 3
    cb_n = C // _L
    rows_per_tile = (N // _NS) // 8 * 8
    rows_tail = N - rows_per_tile * _NS

    mesh = plsc.VectorSubcoreMesh(core_axis_name="c", subcore_axis_name="s")

    @functools.partial(
        pl.kernel,
        out_type=[
            jax.ShapeDtypeStruct((_NC, N, C), jnp.float32),
            jax.ShapeDtypeStruct((_NC, 8, N), jnp.float32),
        ],
        mesh=mesh,
        scratch_types=[
            [pltpu.VMEM((CHUNK,), jnp.int32)] * 2,
            [pltpu.VMEM((CHUNK,), jnp.int32)] * 2,
            [pltpu.VMEM((CHUNK,), jnp.int32)] * 2,
            [pltpu.VMEM((CHUNK, C), jnp.float32)] * 2,
            [pltpu.VMEM((CHUNK, C), jnp.float32)] * 2,
            [pltpu.VMEM((CHUNK,), jnp.float32)] * 2,
            pltpu.VMEM((C,), jnp.float32),
            pltpu.VMEM((_L, 32), jnp.float32),
            pltpu.VMEM_SHARED((N, C), jnp.float32),
            pltpu.VMEM_SHARED((N,), jnp.float32),
            [pltpu.SemaphoreType.DMA] * 2,
            [pltpu.SemaphoreType.DMA] * 2,
            [pltpu.SemaphoreType.DMA] * 2,
        ],
        compiler_params=pltpu.CompilerParams(use_tc_tiling_on_sc=False),
    )
    def k(xl_hbm, xr_hbm, src_hbm, dst_hbm, att_hbm,
          acc_out, den_out,
          sidx2, didx2, didxs2, srows2, drows2, exbuf2, attv, tb, acc_sh,
          den_sh, isem, gsem, ssem):
        cid = lax.axis_index("c")
        sid = lax.axis_index("s")
        w = cid * _NS + sid

        pltpu.sync_copy(att_hbm, attv)

        zbuf = drows2[0]

        def _zero_row(i, _):
            zbuf[i, :] = jnp.zeros((C,), jnp.float32)
            return 0
        lax.fori_loop(0, CHUNK, _zero_row, 0)
        row0 = sid * rows_per_tile
        nfull = rows_per_tile // CHUNK
        rem = rows_per_tile - nfull * CHUNK
        for i in range(nfull):
            pltpu.sync_copy(zbuf, acc_sh.at[pl.ds(row0 + i * CHUNK, CHUNK)])
        if rem:
            pltpu.sync_copy(zbuf.at[pl.ds(0, rem)],
                            acc_sh.at[pl.ds(row0 + nfull * CHUNK, rem)])

        @pl.when(sid == _NS - 1)
        def _():
            pltpu.sync_copy(zbuf.at[pl.ds(0, rows_tail)],
                            acc_sh.at[pl.ds(rows_per_tile * _NS, rows_tail)])

        zex = exbuf2[0]
        zex[...] = jnp.zeros((CHUNK,), jnp.float32)

        @pl.when(sid == 0)
        def _():
            for i in range(N // CHUNK):
                pltpu.sync_copy(zex, den_sh.at[pl.ds(i * CHUNK, CHUNK)])
            dr = N - (N // CHUNK) * CHUNK
            if dr:
                pltpu.sync_copy(zex.at[pl.ds(0, dr)],
                                den_sh.at[pl.ds((N // CHUNK) * CHUNK, dr)])

        plsc.subcore_barrier()

        att_regs = [attv[pl.ds(cb * _L, _L)] for cb in range(cb_n)]
        fiota = lax.iota(jnp.int32, _L).astype(jnp.float32)

        def idx_src(t):
            base = w * per_w + t * CHUNK
            return (src_hbm.at[pl.ds(base, CHUNK)], dst_hbm.at[pl.ds(base, CHUNK)])

        def issue_idx(t, p):
            s_src, d_src = idx_src(t)
            pltpu.async_copy(s_src, sidx2[p], isem[p])
            pltpu.async_copy(d_src, didx2[p], isem[p])

        def wait_idx(t, p):
            s_src, d_src = idx_src(t)
            pltpu.make_async_copy(s_src, sidx2[p], isem[p]).wait()
            pltpu.make_async_copy(d_src, didx2[p], isem[p]).wait()

        def issue_gather(p):
            pltpu.async_copy(xl_hbm.at[sidx2[p]], srows2[p], gsem[p])
            pltpu.async_copy(xr_hbm.at[didx2[p]], drows2[p], gsem[p])

        def wait_gather(p):
            pltpu.make_async_copy(xl_hbm.at[sidx2[p]], srows2[p], gsem[p]).wait()
            pltpu.make_async_copy(xr_hbm.at[didx2[p]], drows2[p], gsem[p]).wait()

        def issue_scatter(p):
            pltpu.async_copy(drows2[p], acc_sh.at[didxs2[p]], ssem[p], add=True)
            pltpu.async_copy(exbuf2[p], den_sh.at[didxs2[p]], ssem[p], add=True)

        def wait_scatter(p):
            pltpu.make_async_copy(drows2[p], acc_sh.at[didxs2[p]], ssem[p]).wait()
            pltpu.make_async_copy(exbuf2[p], den_sh.at[didxs2[p]], ssem[p]).wait()

        def snap_didx(p):
            for q in range(CHUNK // _L):
                didxs2[p][pl.ds(q * _L, _L)] = didx2[p][pl.ds(q * _L, _L)]

        def compute(t, p):
            srows, drows, exbuf = srows2[p], drows2[p], exbuf2[p]
            base = w * per_w + t * CHUNK
            def group_body(g, _):
                e0 = g * _L
                ex_acc = jnp.zeros((_L,), jnp.float32)
                for jj in range(_L):
                    e = e0 + jj
                    acc = jnp.zeros((_L,), jnp.float32)
                    for cb in range(cb_n):
                        u = srows[e, pl.ds(cb * _L, _L)] + drows[e, pl.ds(cb * _L, _L)]
                        lr = jnp.maximum(u, 0.2 * u)
                        acc = acc + lr * att_regs[cb]
                    tb[jj, pl.ds(0, _L)] = acc
                    acc = acc + tb[jj, pl.ds(8, _L)]
                    parts = [acc[c] for c in range(8)]
                    while len(parts) > 1:
                        parts = [a + b for a, b in zip(parts[::2], parts[1::2])]
                    logit = jnp.where((base + e) < E_tot, parts[0], -1e30)
                    exv = jnp.exp(jnp.full((_L,), logit, jnp.float32))
                    for cb in range(cb_n):
                        drows[e, pl.ds(cb * _L, _L)] = srows[e, pl.ds(cb * _L, _L)] * exv
                    onehot = jnp.maximum(
                        1.0 - jnp.abs(fiota - float(jj)), 0.0)
                    ex_acc = ex_acc + exv * onehot
                exbuf[pl.ds(e0, _L)] = ex_acc
                return 0
            lax.fori_loop(0, CHUNK // _L, group_body, 0)

        issue_idx(0, 0)
        issue_idx(1, 1)
        wait_idx(0, 0)
        wait_idx(1, 1)
        issue_gather(0)
        issue_gather(1)
        wait_gather(0)
        snap_didx(0)
        issue_idx(2, 0)
        compute(0, 0)
        issue_scatter(0)

        def loop_body(i, _):
            for b in range(2):
                t = 1 + 2 * i + b
                p = (1 + b) % 2

                @pl.when(t < n_chunks - 1)
                def _():
                    wait_idx(t + 1, 1 - p)
                wait_scatter(1 - p)

                @pl.when(t < n_chunks - 1)
                def _():
                    issue_gather(1 - p)
                wait_gather(p)
                snap_didx(p)

                @pl.when(t < n_chunks - 2)
                def _():
                    issue_idx(t + 2, p)
                compute(t, p)
                issue_scatter(p)
            return 0

        lax.fori_loop(0, (n_chunks - 1) // 2, loop_body, 0)
        wait_scatter(0)

        plsc.subcore_barrier()

        pltpu.sync_copy(acc_sh.at[pl.ds(row0, rows_per_tile)],
                        acc_out.at[cid, pl.ds(row0, rows_per_tile)])

        @pl.when(sid == _NS - 1)
        def _():
            pltpu.sync_copy(acc_sh.at[pl.ds(rows_per_tile * _NS, rows_tail)],
                            acc_out.at[cid, pl.ds(rows_per_tile * _NS, rows_tail)])

        @pl.when(sid == 0)
        def _():
            pltpu.sync_copy(den_sh, den_out.at[cid, 0])

    return k


def _decode_sc(C, N, E_tot, E_pad):
    per_w = E_pad // _NW
    n_chunks = per_w // _CHUNK
    assert n_chunks % 2 == 1 and n_chunks >= 3
    cb_n = C // _L

    mesh = plsc.VectorSubcoreMesh(core_axis_name="c", subcore_axis_name="s")

    @functools.partial(
        pl.kernel,
        out_type=jax.ShapeDtypeStruct((E_pad,), jnp.float32),
        mesh=mesh,
        scratch_types=[
            pltpu.VMEM((n_chunks, _CHUNK), jnp.int32),
            pltpu.VMEM((n_chunks, _CHUNK), jnp.int32),
            [pltpu.VMEM((_CHUNK, C), jnp.float32)] * 2,
            [pltpu.VMEM((_CHUNK, C), jnp.float32)] * 2,
            [pltpu.VMEM((_CHUNK,), jnp.float32)] * 2,
            pltpu.VMEM((_L,), jnp.float32),
            pltpu.VMEM((_L, 32), jnp.float32),
            [pltpu.SemaphoreType.DMA] * 2,
            [pltpu.SemaphoreType.DMA] * 2,
        ],
        compiler_params=pltpu.CompilerParams(use_tc_tiling_on_sc=False),
    )
    def k(g_hbm, h_hbm, src_hbm, dst_hbm, bps_hbm, res_out,
          sidx2, didx2, grows2, hrows2, resbuf2, bpsv, tb, gsem, rsem):
        cid = lax.axis_index("c")
        sid = lax.axis_index("s")
        w = cid * _NS + sid

        pltpu.sync_copy(bps_hbm, bpsv)
        bps = bpsv[...]
        fiota = lax.iota(jnp.int32, _L).astype(jnp.float32)

        pltpu.sync_copy(src_hbm.at[pl.ds(w * n_chunks, n_chunks)], sidx2)
        pltpu.sync_copy(dst_hbm.at[pl.ds(w * n_chunks, n_chunks)], didx2)

        def issue_gather(t, p):
            pltpu.async_copy(g_hbm.at[sidx2.at[t]], grows2[p], gsem[p])
            pltpu.async_copy(h_hbm.at[didx2.at[t]], hrows2[p], gsem[p])

        def wait_gather(t, p):
            pltpu.make_async_copy(g_hbm.at[sidx2.at[t]], grows2[p], gsem[p]).wait()
            pltpu.make_async_copy(h_hbm.at[didx2.at[t]], hrows2[p], gsem[p]).wait()

        def out_ref(t):
            base = w * per_w + t * _CHUNK
            return res_out.at[pl.ds(pl.multiple_of(base, _CHUNK), _CHUNK)]

        def issue_store(t, p):
            pltpu.async_copy(resbuf2[p], out_ref(t), rsem[p])

        def wait_store(t, p):
            pltpu.make_async_copy(resbuf2[p], out_ref(t), rsem[p]).wait()

        def compute(t, p):
            grows, hrows, resbuf = grows2[p], hrows2[p], resbuf2[p]

            def group_body(g, _):
                e0 = g * _L
                r_acc = bps
                for jj in range(_L):
                    e = e0 + jj
                    acc = jnp.zeros((_L,), jnp.float32)
                    for cb in range(cb_n):
                        acc = acc + (grows[e, pl.ds(cb * _L, _L)]
                                     * hrows[e, pl.ds(cb * _L, _L)])
                    tb[jj, pl.ds(0, _L)] = acc
                    acc = acc + tb[jj, pl.ds(8, _L)]
                    parts = [acc[c] for c in range(8)]
                    while len(parts) > 1:
                        parts = [a + b for a, b in zip(parts[::2], parts[1::2])]
                    rv = jnp.full((_L,), parts[0], jnp.float32)
                    onehot = jnp.maximum(
                        1.0 - jnp.abs(fiota - float(jj)), 0.0)
                    r_acc = r_acc + rv * onehot
                resbuf[pl.ds(e0, _L)] = r_acc
                return 0
            lax.fori_loop(0, _CHUNK // _L, group_body, 0)

        issue_gather(0, 0)
        issue_gather(1, 1)
        wait_gather(0, 0)
        compute(0, 0)
        issue_store(0, 0)

        def loop_body(i, _):
            for b in range(2):
                t = 1 + 2 * i + b
                p = (1 + b) % 2

                @pl.when(t + 1 < n_chunks)
                def _():
                    issue_gather(t + 1, 1 - p)
                wait_gather(t, p)
                compute(t, p)
                wait_store(t - 1, 1 - p)
                issue_store(t, p)
            return 0

        lax.fori_loop(0, (n_chunks - 1) // 2, loop_body, 0)
        wait_store(n_chunks - 1, 0)

    return k


def _pad1d(a, n):
    return jnp.concatenate([a, jnp.zeros((n - a.shape[0],), a.dtype)])


def _padded_len(e, gran):
    k = (e + gran - 1) // gran
    if k % 2 == 0:
        k += 1
    return k * gran


def kernel(x, edge_index, edge_label_index, Wl1, bl1, Wr1, br1, att1, bias1,
           Wl2, bl2, Wr2, br2, att2, bias2, Wp, bp):
    N = x.shape[0]
    E = edge_index.shape[1]
    E_tot = E + N
    E_pad1 = _padded_len(E_tot, _NW * 64)
    E_pad2 = _padded_len(E_tot, _NW * 128)
    EL = edge_label_index.shape[1]
    EL_pad = _padded_len(EL, _NW * _CHUNK)

    loop = jnp.arange(N, dtype=edge_index.dtype)
    src_raw = jnp.concatenate([edge_index[0], loop])
    dst_raw = jnp.concatenate([edge_index[1], loop])
    src1, dst1 = _pad1d(src_raw, E_pad1), _pad1d(dst_raw, E_pad1)
    src2, dst2 = _pad1d(src_raw, E_pad2), _pad1d(dst_raw, E_pad2)

    xl1, xr1 = _mm2(x, Wl1, bl1, Wr1, br1)
    acc1, den1 = _edge_sc(128, N, E_tot, E_pad1, 64)(xl1, xr1, src1, dst1, att1)
    xl2, xr2 = _norm_mm2(acc1, den1[:, 0, :].T, bias1, Wl2, bl2, Wr2, br2)

    acc2, den2 = _edge_sc(64, N, E_tot, E_pad2, 128)(xl2, xr2, src2, dst2, att2)
    wpv = Wp @ jnp.ones((2,), jnp.float32)
    h2, g2 = _norm_final(acc2, den2[:, 0, :].T, bias2, wpv)

    srcL = _pad1d(edge_label_index[0], EL_pad).reshape(-1, _CHUNK)
    dstL = _pad1d(edge_label_index[1], EL_pad).reshape(-1, _CHUNK)
    bps = jnp.full((_L,), jnp.sum(bp), jnp.float32)
    res = _decode_sc(64, N, EL, EL_pad)(g2, h2, srcL, dstL, bps)
    return res[:EL]


# speedup vs baseline: 1.3457x; 1.3444x over previous
import functools

import jax
import jax.numpy as jnp
from jax import lax
from jax.experimental import pallas as pl
from jax.experimental.pallas import tpu as pltpu
from jax.experimental.pallas import tpu_sc as plsc

_L = 16
_NC = 2
_NS = 16
_NW = _NC * _NS
_CHUNK = 128


def _mm2_body(x_ref, wl_ref, bl_ref, wr_ref, br_ref, xl_ref, xr_ref):
    x = x_ref[...]
    xl_ref[...] = jnp.dot(x, wl_ref[...], preferred_element_type=jnp.float32) + bl_ref[...]
    xr_ref[...] = jnp.dot(x, wr_ref[...], preferred_element_type=jnp.float32) + br_ref[...]


def _mm2(x, Wl, bl, Wr, br):
    N, Din = x.shape
    H = Wl.shape[1]
    blk = 400
    return pl.pallas_call(
        _mm2_body,
        grid=(N // blk,),
        in_specs=[
            pl.BlockSpec((blk, Din), lambda i: (i, 0)),
            pl.BlockSpec((Din, H), lambda i: (0, 0)),
            pl.BlockSpec((1, H), lambda i: (0, 0)),
            pl.BlockSpec((Din, H), lambda i: (0, 0)),
            pl.BlockSpec((1, H), lambda i: (0, 0)),
        ],
        out_specs=[
            pl.BlockSpec((blk, H), lambda i: (i, 0)),
            pl.BlockSpec((blk, H), lambda i: (i, 0)),
        ],
        out_shape=[
            jax.ShapeDtypeStruct((N, H), jnp.float32),
            jax.ShapeDtypeStruct((N, H), jnp.float32),
        ],
    )(x, Wl, bl.reshape(1, H), Wr, br.reshape(1, H))


def _norm_mm2_body(acc_ref, den_ref, b_ref, wl_ref, bl_ref, wr_ref, br_ref,
                   xl_ref, xr_ref):
    acc = acc_ref[0] + acc_ref[1]
    den = den_ref[:, 0:1] + den_ref[:, 1:2]
    h = jnp.maximum(acc / (den + 1e-16) + b_ref[...], 0.0)
    xl_ref[...] = jnp.dot(h, wl_ref[...], preferred_element_type=jnp.float32) + bl_ref[...]
    xr_ref[...] = jnp.dot(h, wr_ref[...], preferred_element_type=jnp.float32) + br_ref[...]


def _norm_mm2(acc_parts, denT, bias, Wl, bl, Wr, br):
    _, N, C = acc_parts.shape
    H = Wl.shape[1]
    blk = 400
    return pl.pallas_call(
        _norm_mm2_body,
        grid=(N // blk,),
        in_specs=[
            pl.BlockSpec((2, blk, C), lambda i: (0, i, 0)),
            pl.BlockSpec((blk, 2), lambda i: (i, 0)),
            pl.BlockSpec((1, C), lambda i: (0, 0)),
            pl.BlockSpec((C, H), lambda i: (0, 0)),
            pl.BlockSpec((1, H), lambda i: (0, 0)),
            pl.BlockSpec((C, H), lambda i: (0, 0)),
            pl.BlockSpec((1, H), lambda i: (0, 0)),
        ],
        out_specs=[
            pl.BlockSpec((blk, H), lambda i: (i, 0)),
            pl.BlockSpec((blk, H), lambda i: (i, 0)),
        ],
        out_shape=[
            jax.ShapeDtypeStruct((N, H), jnp.float32),
            jax.ShapeDtypeStruct((N, H), jnp.float32),
        ],
    )(acc_parts, denT, bias.reshape(1, C), Wl, bl.reshape(1, H), Wr, br.reshape(1, H))


def _norm_final_body(acc_ref, den_ref, b_ref, wpv_ref, h_ref, g_ref):
    acc = acc_ref[0] + acc_ref[1]
    den = den_ref[:, 0:1] + den_ref[:, 1:2]
    h = jnp.maximum(acc / (den + 1e-16) + b_ref[...], 0.0)
    h_ref[...] = h
    g_ref[...] = h * wpv_ref[...]


def _norm_final(acc_parts, denT, bias, wpv):
    _, N, C = acc_parts.shape
    blk = 400
    return pl.pallas_call(
        _norm_final_body,
        grid=(N // blk,),
        in_specs=[
            pl.BlockSpec((2, blk, C), lambda i: (0, i, 0)),
            pl.BlockSpec((blk, 2), lambda i: (i, 0)),
            pl.BlockSpec((1, C), lambda i: (0, 0)),
            pl.BlockSpec((1, C), lambda i: (0, 0)),
        ],
        out_specs=[
            pl.BlockSpec((blk, C), lambda i: (i, 0)),
            pl.BlockSpec((blk, C), lambda i: (i, 0)),
        ],
        out_shape=[
            jax.ShapeDtypeStruct((N, C), jnp.float32),
            jax.ShapeDtypeStruct((N, C), jnp.float32),
        ],
    )(acc_parts, denT, bias.reshape(1, C), wpv.reshape(1, C))


def _edge_sc(C, N, E_tot, E_pad, CHUNK):
    per_w = E_pad // _NW
    n_chunks = per_w // CHUNK
    assert n_chunks % 2 == 1 and n_chunks >= 3
    cb_n = C // _L
    rows_per_tile = (N // _NS) // 8 * 8
    rows_tail = N - rows_per_tile * _NS

    mesh = plsc.VectorSubcoreMesh(core_axis_name="c", subcore_axis_name="s")

    @functools.partial(
        pl.kernel,
        out_type=[
            jax.ShapeDtypeStruct((_NC, N, C), jnp.float32),
            jax.ShapeDtypeStruct((_NC, 8, N), jnp.float32),
        ],
        mesh=mesh,
        scratch_types=[
            [pltpu.VMEM((CHUNK,), jnp.int32)] * 2,
            [pltpu.VMEM((CHUNK,), jnp.int32)] * 2,
            [pltpu.VMEM((CHUNK,), jnp.int32)] * 2,
            [pltpu.VMEM((CHUNK, C), jnp.float32)] * 2,
            [pltpu.VMEM((CHUNK, C), jnp.float32)] * 2,
            [pltpu.VMEM((CHUNK,), jnp.float32)] * 2,
            pltpu.VMEM((C,), jnp.float32),
            pltpu.VMEM((_L, 32), jnp.float32),
            pltpu.VMEM_SHARED((N, C), jnp.float32),
            pltpu.VMEM_SHARED((N,), jnp.float32),
            [pltpu.SemaphoreType.DMA] * 2,
            [pltpu.SemaphoreType.DMA] * 2,
            [pltpu.SemaphoreType.DMA] * 2,
        ],
        compiler_params=pltpu.CompilerParams(use_tc_tiling_on_sc=False),
    )
    def k(xl_hbm, xr_hbm, src_hbm, dst_hbm, att_hbm,
          acc_out, den_out,
          sidx2, didx2, didxs2, srows2, drows2, exbuf2, attv, tb, acc_sh,
          den_sh, isem, gsem, ssem):
        cid = lax.axis_index("c")
        sid = lax.axis_index("s")
        w = cid * _NS + sid

        pltpu.sync_copy(att_hbm, attv)

        zbuf = drows2[0]

        def _zero_row(i, _):
            zbuf[i, :] = jnp.zeros((C,), jnp.float32)
            return 0
        lax.fori_loop(0, CHUNK, _zero_row, 0)
        row0 = sid * rows_per_tile
        nfull = rows_per_tile // CHUNK
        rem = rows_per_tile - nfull * CHUNK
        for i in range(nfull):
            pltpu.sync_copy(zbuf, acc_sh.at[pl.ds(row0 + i * CHUNK, CHUNK)])
        if rem:
            pltpu.sync_copy(zbuf.at[pl.ds(0, rem)],
                            acc_sh.at[pl.ds(row0 + nfull * CHUNK, rem)])

        @pl.when(sid == _NS - 1)
        def _():
            pltpu.sync_copy(zbuf.at[pl.ds(0, rows_tail)],
                            acc_sh.at[pl.ds(rows_per_tile * _NS, rows_tail)])

        zex = exbuf2[0]
        zex[...] = jnp.zeros((CHUNK,), jnp.float32)

        @pl.when(sid == 0)
        def _():
            for i in range(N // CHUNK):
                pltpu.sync_copy(zex, den_sh.at[pl.ds(i * CHUNK, CHUNK)])
            dr = N - (N // CHUNK) * CHUNK
            if dr:
                pltpu.sync_copy(zex.at[pl.ds(0, dr)],
                                den_sh.at[pl.ds((N // CHUNK) * CHUNK, dr)])

        plsc.subcore_barrier()

        att_regs = [attv[pl.ds(cb * _L, _L)] for cb in range(cb_n)]
        fiota = lax.iota(jnp.int32, _L).astype(jnp.float32)

        def idx_src(t):
            base = w * per_w + t * CHUNK
            return (src_hbm.at[pl.ds(base, CHUNK)], dst_hbm.at[pl.ds(base, CHUNK)])

        def issue_idx(t, p):
            s_src, d_src = idx_src(t)
            pltpu.async_copy(s_src, sidx2[p], isem[p])
            pltpu.async_copy(d_src, didx2[p], isem[p])

        def wait_idx(t, p):
            s_src, d_src = idx_src(t)
            pltpu.make_async_copy(s_src, sidx2[p], isem[p]).wait()
            pltpu.make_async_copy(d_src, didx2[p], isem[p]).wait()

        def issue_gather(p):
            pltpu.async_copy(xl_hbm.at[sidx2[p]], srows2[p], gsem[p])
            pltpu.async_copy(xr_hbm.at[didx2[p]], drows2[p], gsem[p])

        def wait_gather(p):
            pltpu.make_async_copy(xl_hbm.at[sidx2[p]], srows2[p], gsem[p]).wait()
            pltpu.make_async_copy(xr_hbm.at[didx2[p]], drows2[p], gsem[p]).wait()

        def issue_scatter(p):
            pltpu.async_copy(drows2[p], acc_sh.at[didxs2[p]], ssem[p], add=True)
            pltpu.async_copy(exbuf2[p], den_sh.at[didxs2[p]], ssem[p], add=True)

        def wait_scatter(p):
            pltpu.make_async_copy(drows2[p], acc_sh.at[didxs2[p]], ssem[p]).wait()
            pltpu.make_async_copy(exbuf2[p], den_sh.at[didxs2[p]], ssem[p]).wait()

        def snap_didx(p):
            for q in range(CHUNK // _L):
                didxs2[p][pl.ds(q * _L, _L)] = didx2[p][pl.ds(q * _L, _L)]

        def compute(t, p):
            srows, drows, exbuf = srows2[p], drows2[p], exbuf2[p]
            base = w * per_w + t * CHUNK
            def group_body(g, _):
                e0 = g * _L
                ex_acc = jnp.zeros((_L,), jnp.float32)
                for jj in range(_L):
                    e = e0 + jj
                    acc = jnp.zeros((_L,), jnp.float32)
                    for cb in range(cb_n):
                        u = srows[e, pl.ds(cb * _L, _L)] + drows[e, pl.ds(cb * _L, _L)]
                        lr = jnp.maximum(u, 0.2 * u)
                        acc = acc + lr * att_regs[cb]
                    parts = [acc[c] for c in range(_L)]
                    while len(parts) > 1:
                        parts = [a + b for a, b in zip(parts[::2], parts[1::2])]
                    logit = jnp.where((base + e) < E_tot, parts[0], -1e30)
                    exv = jnp.exp(jnp.full((_L,), logit, jnp.float32))
                    for cb in range(cb_n):
                        drows[e, pl.ds(cb * _L, _L)] = srows[e, pl.ds(cb * _L, _L)] * exv
                    onehot = jnp.maximum(
                        1.0 - jnp.abs(fiota - float(jj)), 0.0)
                    ex_acc = ex_acc + exv * onehot
                exbuf[pl.ds(e0, _L)] = ex_acc
                return 0
            lax.fori_loop(0, CHUNK // _L, group_body, 0)

        issue_idx(0, 0)
        issue_idx(1, 1)
        wait_idx(0, 0)
        wait_idx(1, 1)
        issue_gather(0)
        issue_gather(1)
        wait_gather(0)
        snap_didx(0)
        issue_idx(2, 0)
        compute(0, 0)
        issue_scatter(0)

        def loop_body(i, _):
            for b in range(2):
                t = 1 + 2 * i + b
                p = (1 + b) % 2

                @pl.when(t < n_chunks - 1)
                def _():
                    wait_idx(t + 1, 1 - p)
                wait_scatter(1 - p)

                @pl.when(t < n_chunks - 1)
                def _():
                    issue_gather(1 - p)
                wait_gather(p)
                snap_didx(p)

                @pl.when(t < n_chunks - 2)
                def _():
                    issue_idx(t + 2, p)
                compute(t, p)
                issue_scatter(p)
            return 0

        lax.fori_loop(0, (n_chunks - 1) // 2, loop_body, 0)
        wait_scatter(0)

        plsc.subcore_barrier()

        pltpu.sync_copy(acc_sh.at[pl.ds(row0, rows_per_tile)],
                        acc_out.at[cid, pl.ds(row0, rows_per_tile)])

        @pl.when(sid == _NS - 1)
        def _():
            pltpu.sync_copy(acc_sh.at[pl.ds(rows_per_tile * _NS, rows_tail)],
                            acc_out.at[cid, pl.ds(rows_per_tile * _NS, rows_tail)])

        @pl.when(sid == 0)
        def _():
            pltpu.sync_copy(den_sh, den_out.at[cid, 0])

    return k


def _decode_sc(C, N, E_tot, E_pad):
    per_w = E_pad // _NW
    n_chunks = per_w // _CHUNK
    assert n_chunks % 2 == 1 and n_chunks >= 3
    cb_n = C // _L

    mesh = plsc.VectorSubcoreMesh(core_axis_name="c", subcore_axis_name="s")

    @functools.partial(
        pl.kernel,
        out_type=jax.ShapeDtypeStruct((E_pad,), jnp.float32),
        mesh=mesh,
        scratch_types=[
            pltpu.VMEM((n_chunks, _CHUNK), jnp.int32),
            pltpu.VMEM((n_chunks, _CHUNK), jnp.int32),
            [pltpu.VMEM((_CHUNK, C), jnp.float32)] * 2,
            [pltpu.VMEM((_CHUNK, C), jnp.float32)] * 2,
            [pltpu.VMEM((_CHUNK,), jnp.float32)] * 2,
            pltpu.VMEM((_L,), jnp.float32),
            pltpu.VMEM((_L, 32), jnp.float32),
            [pltpu.SemaphoreType.DMA] * 2,
            [pltpu.SemaphoreType.DMA] * 2,
        ],
        compiler_params=pltpu.CompilerParams(use_tc_tiling_on_sc=False),
    )
    def k(g_hbm, h_hbm, src_hbm, dst_hbm, bps_hbm, res_out,
          sidx2, didx2, grows2, hrows2, resbuf2, bpsv, tb, gsem, rsem):
        cid = lax.axis_index("c")
        sid = lax.axis_index("s")
        w = cid * _NS + sid

        pltpu.sync_copy(bps_hbm, bpsv)
        bps = bpsv[...]
        fiota = lax.iota(jnp.int32, _L).astype(jnp.float32)

        pltpu.sync_copy(src_hbm.at[pl.ds(w * n_chunks, n_chunks)], sidx2)
        pltpu.sync_copy(dst_hbm.at[pl.ds(w * n_chunks, n_chunks)], didx2)

        def issue_gather(t, p):
            pltpu.async_copy(g_hbm.at[sidx2.at[t]], grows2[p], gsem[p])
            pltpu.async_copy(h_hbm.at[didx2.at[t]], hrows2[p], gsem[p])

        def wait_gather(t, p):
            pltpu.make_async_copy(g_hbm.at[sidx2.at[t]], grows2[p], gsem[p]).wait()
            pltpu.make_async_copy(h_hbm.at[didx2.at[t]], hrows2[p], gsem[p]).wait()

        def out_ref(t):
            base = w * per_w + t * _CHUNK
            return res_out.at[pl.ds(pl.multiple_of(base, _CHUNK), _CHUNK)]

        def issue_store(t, p):
            pltpu.async_copy(resbuf2[p], out_ref(t), rsem[p])

        def wait_store(t, p):
            pltpu.make_async_copy(resbuf2[p], out_ref(t), rsem[p]).wait()

        def compute(t, p):
            grows, hrows, resbuf = grows2[p], hrows2[p], resbuf2[p]

            def group_body(g, _):
                e0 = g * _L
                r_acc = bps
                for jj in range(_L):
                    e = e0 + jj
                    acc = jnp.zeros((_L,), jnp.float32)
                    for cb in range(cb_n):
                        acc = acc + (grows[e, pl.ds(cb * _L, _L)]
                                     * hrows[e, pl.ds(cb * _L, _L)])
                    parts = [acc[c] for c in range(_L)]
                    while len(parts) > 1:
                        parts = [a + b for a, b in zip(parts[::2], parts[1::2])]
                    rv = jnp.full((_L,), parts[0], jnp.float32)
                    onehot = jnp.maximum(
                        1.0 - jnp.abs(fiota - float(jj)), 0.0)
                    r_acc = r_acc + rv * onehot
                resbuf[pl.ds(e0, _L)] = r_acc
                return 0
            lax.fori_loop(0, _CHUNK // _L, group_body, 0)

        issue_gather(0, 0)
        issue_gather(1, 1)
        wait_gather(0, 0)
        compute(0, 0)
        issue_store(0, 0)

        def loop_body(i, _):
            for b in range(2):
                t = 1 + 2 * i + b
                p = (1 + b) % 2

                @pl.when(t + 1 < n_chunks)
                def _():
                    issue_gather(t + 1, 1 - p)
                wait_gather(t, p)
                compute(t, p)
                wait_store(t - 1, 1 - p)
                issue_store(t, p)
            return 0

        lax.fori_loop(0, (n_chunks - 1) // 2, loop_body, 0)
        wait_store(n_chunks - 1, 0)

    return k


def _pad1d(a, n):
    return jnp.concatenate([a, jnp.zeros((n - a.shape[0],), a.dtype)])


def _padded_len(e, gran):
    k = (e + gran - 1) // gran
    if k % 2 == 0:
        k += 1
    return k * gran


def kernel(x, edge_index, edge_label_index, Wl1, bl1, Wr1, br1, att1, bias1,
           Wl2, bl2, Wr2, br2, att2, bias2, Wp, bp):
    N = x.shape[0]
    E = edge_index.shape[1]
    E_tot = E + N
    E_pad1 = _padded_len(E_tot, _NW * 64)
    E_pad2 = _padded_len(E_tot, _NW * 128)
    EL = edge_label_index.shape[1]
    EL_pad = _padded_len(EL, _NW * _CHUNK)

    loop = jnp.arange(N, dtype=edge_index.dtype)
    src_raw = jnp.concatenate([edge_index[0], loop])
    dst_raw = jnp.concatenate([edge_index[1], loop])
    src1, dst1 = _pad1d(src_raw, E_pad1), _pad1d(dst_raw, E_pad1)
    src2, dst2 = _pad1d(src_raw, E_pad2), _pad1d(dst_raw, E_pad2)

    xl1, xr1 = _mm2(x, Wl1, bl1, Wr1, br1)
    acc1, den1 = _edge_sc(128, N, E_tot, E_pad1, 64)(xl1, xr1, src1, dst1, att1)
    xl2, xr2 = _norm_mm2(acc1, den1[:, 0, :].T, bias1, Wl2, bl2, Wr2, br2)

    acc2, den2 = _edge_sc(64, N, E_tot, E_pad2, 128)(xl2, xr2, src2, dst2, att2)
    wpv = Wp @ jnp.ones((2,), jnp.float32)
    h2, g2 = _norm_final(acc2, den2[:, 0, :].T, bias2, wpv)

    srcL = _pad1d(edge_label_index[0], EL_pad).reshape(-1, _CHUNK)
    dstL = _pad1d(edge_label_index[1], EL_pad).reshape(-1, _CHUNK)
    bps = jnp.full((_L,), jnp.sum(bp), jnp.float32)
    res = _decode_sc(64, N, EL, EL_pad)(g2, h2, srcL, dstL, bps)
    return res[:EL]


# speedup vs baseline: 1.3671x; 1.0159x over previous
import functools

import jax
import jax.numpy as jnp
from jax import lax
from jax.experimental import pallas as pl
from jax.experimental.pallas import tpu as pltpu
from jax.experimental.pallas import tpu_sc as plsc

_L = 16
_NC = 2
_NS = 16
_NW = _NC * _NS
_CHUNK = 128


def _mm2_body(x_ref, wl_ref, bl_ref, wr_ref, br_ref, xl_ref, xr_ref):
    x = x_ref[...]
    xl_ref[...] = jnp.dot(x, wl_ref[...], preferred_element_type=jnp.float32) + bl_ref[...]
    xr_ref[...] = jnp.dot(x, wr_ref[...], preferred_element_type=jnp.float32) + br_ref[...]


def _mm2(x, Wl, bl, Wr, br):
    N, Din = x.shape
    H = Wl.shape[1]
    blk = 400
    return pl.pallas_call(
        _mm2_body,
        grid=(N // blk,),
        in_specs=[
            pl.BlockSpec((blk, Din), lambda i: (i, 0)),
            pl.BlockSpec((Din, H), lambda i: (0, 0)),
            pl.BlockSpec((1, H), lambda i: (0, 0)),
            pl.BlockSpec((Din, H), lambda i: (0, 0)),
            pl.BlockSpec((1, H), lambda i: (0, 0)),
        ],
        out_specs=[
            pl.BlockSpec((blk, H), lambda i: (i, 0)),
            pl.BlockSpec((blk, H), lambda i: (i, 0)),
        ],
        out_shape=[
            jax.ShapeDtypeStruct((N, H), jnp.float32),
            jax.ShapeDtypeStruct((N, H), jnp.float32),
        ],
    )(x, Wl, bl.reshape(1, H), Wr, br.reshape(1, H))


def _norm_mm2_body(acc_ref, den_ref, b_ref, wl_ref, bl_ref, wr_ref, br_ref,
                   xl_ref, xr_ref):
    acc = acc_ref[0] + acc_ref[1]
    den = den_ref[:, 0:1] + den_ref[:, 1:2]
    h = jnp.maximum(acc / (den + 1e-16) + b_ref[...], 0.0)
    xl_ref[...] = jnp.dot(h, wl_ref[...], preferred_element_type=jnp.float32) + bl_ref[...]
    xr_ref[...] = jnp.dot(h, wr_ref[...], preferred_element_type=jnp.float32) + br_ref[...]


def _norm_mm2(acc_parts, denT, bias, Wl, bl, Wr, br):
    _, N, C = acc_parts.shape
    H = Wl.shape[1]
    blk = 400
    return pl.pallas_call(
        _norm_mm2_body,
        grid=(N // blk,),
        in_specs=[
            pl.BlockSpec((2, blk, C), lambda i: (0, i, 0)),
            pl.BlockSpec((blk, 2), lambda i: (i, 0)),
            pl.BlockSpec((1, C), lambda i: (0, 0)),
            pl.BlockSpec((C, H), lambda i: (0, 0)),
            pl.BlockSpec((1, H), lambda i: (0, 0)),
            pl.BlockSpec((C, H), lambda i: (0, 0)),
            pl.BlockSpec((1, H), lambda i: (0, 0)),
        ],
        out_specs=[
            pl.BlockSpec((blk, H), lambda i: (i, 0)),
            pl.BlockSpec((blk, H), lambda i: (i, 0)),
        ],
        out_shape=[
            jax.ShapeDtypeStruct((N, H), jnp.float32),
            jax.ShapeDtypeStruct((N, H), jnp.float32),
        ],
    )(acc_parts, denT, bias.reshape(1, C), Wl, bl.reshape(1, H), Wr, br.reshape(1, H))


def _norm_final_body(acc_ref, den_ref, b_ref, wpv_ref, h_ref, g_ref):
    acc = acc_ref[0] + acc_ref[1]
    den = den_ref[:, 0:1] + den_ref[:, 1:2]
    h = jnp.maximum(acc / (den + 1e-16) + b_ref[...], 0.0)
    h_ref[...] = h
    g_ref[...] = h * wpv_ref[...]


def _norm_final(acc_parts, denT, bias, wpv):
    _, N, C = acc_parts.shape
    blk = 400
    return pl.pallas_call(
        _norm_final_body,
        grid=(N // blk,),
        in_specs=[
            pl.BlockSpec((2, blk, C), lambda i: (0, i, 0)),
            pl.BlockSpec((blk, 2), lambda i: (i, 0)),
            pl.BlockSpec((1, C), lambda i: (0, 0)),
            pl.BlockSpec((1, C), lambda i: (0, 0)),
        ],
        out_specs=[
            pl.BlockSpec((blk, C), lambda i: (i, 0)),
            pl.BlockSpec((blk, C), lambda i: (i, 0)),
        ],
        out_shape=[
            jax.ShapeDtypeStruct((N, C), jnp.float32),
            jax.ShapeDtypeStruct((N, C), jnp.float32),
        ],
    )(acc_parts, denT, bias.reshape(1, C), wpv.reshape(1, C))


def _edge_sc(C, N, E_tot, E_pad, CHUNK):
    per_w = E_pad // _NW
    n_chunks = per_w // CHUNK
    assert n_chunks % 2 == 1 and n_chunks >= 3
    cb_n = C // _L
    rows_per_tile = (N // _NS) // 8 * 8
    rows_tail = N - rows_per_tile * _NS

    mesh = plsc.VectorSubcoreMesh(core_axis_name="c", subcore_axis_name="s")

    @functools.partial(
        pl.kernel,
        out_type=[
            jax.ShapeDtypeStruct((_NC, N, C), jnp.float32),
            jax.ShapeDtypeStruct((_NC, 8, N), jnp.float32),
        ],
        mesh=mesh,
        scratch_types=[
            [pltpu.VMEM((CHUNK,), jnp.int32)] * 2,
            [pltpu.VMEM((CHUNK,), jnp.int32)] * 2,
            [pltpu.VMEM((CHUNK,), jnp.int32)] * 2,
            [pltpu.VMEM((CHUNK, C), jnp.float32)] * 2,
            [pltpu.VMEM((CHUNK, C), jnp.float32)] * 2,
            [pltpu.VMEM((CHUNK,), jnp.float32)] * 2,
            pltpu.VMEM((C,), jnp.float32),
            pltpu.VMEM((640,), jnp.float32),
            pltpu.VMEM_SHARED((N, C), jnp.float32),
            pltpu.VMEM_SHARED((N,), jnp.float32),
            [pltpu.SemaphoreType.DMA] * 2,
            [pltpu.SemaphoreType.DMA] * 2,
            [pltpu.SemaphoreType.DMA] * 2,
        ],
        compiler_params=pltpu.CompilerParams(use_tc_tiling_on_sc=False),
    )
    def k(xl_hbm, xr_hbm, src_hbm, dst_hbm, att_hbm,
          acc_out, den_out,
          sidx2, didx2, didxs2, srows2, drows2, exbuf2, attv, zden, acc_sh,
          den_sh, isem, gsem, ssem):
        cid = lax.axis_index("c")
        sid = lax.axis_index("s")
        w = cid * _NS + sid

        pltpu.sync_copy(att_hbm, attv)

        zbuf = drows2[0]

        def _zero_row(i, _):
            zbuf[i, :] = jnp.zeros((C,), jnp.float32)
            return 0
        lax.fori_loop(0, CHUNK, _zero_row, 0)
        row0 = sid * rows_per_tile
        nfull = rows_per_tile // CHUNK
        rem = rows_per_tile - nfull * CHUNK
        for i in range(nfull):
            pltpu.sync_copy(zbuf, acc_sh.at[pl.ds(row0 + i * CHUNK, CHUNK)])
        if rem:
            pltpu.sync_copy(zbuf.at[pl.ds(0, rem)],
                            acc_sh.at[pl.ds(row0 + nfull * CHUNK, rem)])

        @pl.when(sid == _NS - 1)
        def _():
            pltpu.sync_copy(zbuf.at[pl.ds(0, rows_tail)],
                            acc_sh.at[pl.ds(rows_per_tile * _NS, rows_tail)])

        def _zden_row(i, _):
            zden[pl.ds(i * _L, _L)] = jnp.zeros((_L,), jnp.float32)
            return 0
        lax.fori_loop(0, 640 // _L, _zden_row, 0)
        pltpu.sync_copy(zden.at[pl.ds(0, rows_per_tile)],
                        den_sh.at[pl.ds(row0, rows_per_tile)])

        @pl.when(sid == _NS - 1)
        def _():
            pltpu.sync_copy(zden.at[pl.ds(0, rows_tail)],
                            den_sh.at[pl.ds(rows_per_tile * _NS, rows_tail)])

        plsc.subcore_barrier()

        att_regs = [attv[pl.ds(cb * _L, _L)] for cb in range(cb_n)]
        fiota = lax.iota(jnp.int32, _L).astype(jnp.float32)

        def idx_src(t):
            base = w * per_w + t * CHUNK
            return (src_hbm.at[pl.ds(base, CHUNK)], dst_hbm.at[pl.ds(base, CHUNK)])

        def issue_idx(t, p):
            s_src, d_src = idx_src(t)
            pltpu.async_copy(s_src, sidx2[p], isem[p])
            pltpu.async_copy(d_src, didx2[p], isem[p])

        def wait_idx(t, p):
            s_src, d_src = idx_src(t)
            pltpu.make_async_copy(s_src, sidx2[p], isem[p]).wait()
            pltpu.make_async_copy(d_src, didx2[p], isem[p]).wait()

        def issue_gather(p):
            pltpu.async_copy(xl_hbm.at[sidx2[p]], srows2[p], gsem[p])
            pltpu.async_copy(xr_hbm.at[didx2[p]], drows2[p], gsem[p])

        def wait_gather(p):
            pltpu.make_async_copy(xl_hbm.at[sidx2[p]], srows2[p], gsem[p]).wait()
            pltpu.make_async_copy(xr_hbm.at[didx2[p]], drows2[p], gsem[p]).wait()

        def issue_scatter(p):
            pltpu.async_copy(drows2[p], acc_sh.at[didxs2[p]], ssem[p], add=True)
            pltpu.async_copy(exbuf2[p], den_sh.at[didxs2[p]], ssem[p], add=True)

        def wait_scatter(p):
            pltpu.make_async_copy(drows2[p], acc_sh.at[didxs2[p]], ssem[p]).wait()
            pltpu.make_async_copy(exbuf2[p], den_sh.at[didxs2[p]], ssem[p]).wait()

        def snap_didx(p):
            for q in range(CHUNK // _L):
                didxs2[p][pl.ds(q * _L, _L)] = didx2[p][pl.ds(q * _L, _L)]

        def compute(t, p):
            srows, drows, exbuf = srows2[p], drows2[p], exbuf2[p]
            base = w * per_w + t * CHUNK
            def group_body(g, _):
                e0 = g * _L
                ex_acc = jnp.zeros((_L,), jnp.float32)
                for jj in range(_L):
                    e = e0 + jj
                    acc = jnp.zeros((_L,), jnp.float32)
                    for cb in range(cb_n):
                        u = srows[e, pl.ds(cb * _L, _L)] + drows[e, pl.ds(cb * _L, _L)]
                        lr = jnp.maximum(u, 0.2 * u)
                        acc = acc + lr * att_regs[cb]
                    parts = [acc[c] for c in range(_L)]
                    while len(parts) > 1:
                        parts = [a + b for a, b in zip(parts[::2], parts[1::2])]
                    logit = jnp.where((base + e) < E_tot, parts[0], -1e30)
                    exv = jnp.exp(jnp.full((_L,), logit, jnp.float32))
                    for cb in range(cb_n):
                        drows[e, pl.ds(cb * _L, _L)] = srows[e, pl.ds(cb * _L, _L)] * exv
                    onehot = jnp.maximum(
                        1.0 - jnp.abs(fiota - float(jj)), 0.0)
                    ex_acc = ex_acc + exv * onehot
                exbuf[pl.ds(e0, _L)] = ex_acc
                return 0
            lax.fori_loop(0, CHUNK // _L, group_body, 0)

        issue_idx(0, 0)
        issue_idx(1, 1)
        wait_idx(0, 0)
        wait_idx(1, 1)
        issue_gather(0)
        issue_gather(1)
        wait_gather(0)
        snap_didx(0)
        issue_idx(2, 0)
        compute(0, 0)
        issue_scatter(0)

        def loop_body(i, _):
            for b in range(2):
                t = 1 + 2 * i + b
                p = (1 + b) % 2

                @pl.when(t < n_chunks - 1)
                def _():
                    wait_idx(t + 1, 1 - p)
                wait_scatter(1 - p)

                @pl.when(t < n_chunks - 1)
                def _():
                    issue_gather(1 - p)
                wait_gather(p)
                snap_didx(p)

                @pl.when(t < n_chunks - 2)
                def _():
                    issue_idx(t + 2, p)
                compute(t, p)
                issue_scatter(p)
            return 0

        lax.fori_loop(0, (n_chunks - 1) // 2, loop_body, 0)
        wait_scatter(0)

        plsc.subcore_barrier()

        pltpu.sync_copy(acc_sh.at[pl.ds(row0, rows_per_tile)],
                        acc_out.at[cid, pl.ds(row0, rows_per_tile)])

        @pl.when(sid == _NS - 1)
        def _():
            pltpu.sync_copy(acc_sh.at[pl.ds(rows_per_tile * _NS, rows_tail)],
                            acc_out.at[cid, pl.ds(rows_per_tile * _NS, rows_tail)])

        @pl.when(sid == 0)
        def _():
            pltpu.sync_copy(den_sh, den_out.at[cid, 0])

    return k


def _decode_sc(C, N, E_tot, E_pad):
    per_w = E_pad // _NW
    n_chunks = per_w // _CHUNK
    assert n_chunks % 2 == 1 and n_chunks >= 3
    cb_n = C // _L

    mesh = plsc.VectorSubcoreMesh(core_axis_name="c", subcore_axis_name="s")

    @functools.partial(
        pl.kernel,
        out_type=jax.ShapeDtypeStruct((E_pad,), jnp.float32),
        mesh=mesh,
        scratch_types=[
            pltpu.VMEM((n_chunks, _CHUNK), jnp.int32),
            pltpu.VMEM((n_chunks, _CHUNK), jnp.int32),
            [pltpu.VMEM((_CHUNK, C), jnp.float32)] * 2,
            [pltpu.VMEM((_CHUNK, C), jnp.float32)] * 2,
            [pltpu.VMEM((_CHUNK,), jnp.float32)] * 2,
            pltpu.VMEM((_L,), jnp.float32),
            [pltpu.SemaphoreType.DMA] * 2,
            [pltpu.SemaphoreType.DMA] * 2,
        ],
        compiler_params=pltpu.CompilerParams(use_tc_tiling_on_sc=False),
    )
    def k(g_hbm, h_hbm, src_hbm, dst_hbm, bps_hbm, res_out,
          sidx2, didx2, grows2, hrows2, resbuf2, bpsv, gsem, rsem):
        cid = lax.axis_index("c")
        sid = lax.axis_index("s")
        w = cid * _NS + sid

        pltpu.sync_copy(bps_hbm, bpsv)
        bps = bpsv[...]
        fiota = lax.iota(jnp.int32, _L).astype(jnp.float32)

        pltpu.sync_copy(src_hbm.at[pl.ds(w * n_chunks, n_chunks)], sidx2)
        pltpu.sync_copy(dst_hbm.at[pl.ds(w * n_chunks, n_chunks)], didx2)

        def issue_gather(t, p):
            pltpu.async_copy(g_hbm.at[sidx2.at[t]], grows2[p], gsem[p])
            pltpu.async_copy(h_hbm.at[didx2.at[t]], hrows2[p], gsem[p])

        def wait_gather(t, p):
            pltpu.make_async_copy(g_hbm.at[sidx2.at[t]], grows2[p], gsem[p]).wait()
            pltpu.make_async_copy(h_hbm.at[didx2.at[t]], hrows2[p], gsem[p]).wait()

        def out_ref(t):
            base = w * per_w + t * _CHUNK
            return res_out.at[pl.ds(pl.multiple_of(base, _CHUNK), _CHUNK)]

        def issue_store(t, p):
            pltpu.async_copy(resbuf2[p], out_ref(t), rsem[p])

        def wait_store(t, p):
            pltpu.make_async_copy(resbuf2[p], out_ref(t), rsem[p]).wait()

        def compute(t, p):
            grows, hrows, resbuf = grows2[p], hrows2[p], resbuf2[p]

            def group_body(g, _):
                e0 = g * _L
                r_acc = bps
                for jj in range(_L):
                    e = e0 + jj
                    acc = jnp.zeros((_L,), jnp.float32)
                    for cb in range(cb_n):
                        acc = acc + (grows[e, pl.ds(cb * _L, _L)]
                                     * hrows[e, pl.ds(cb * _L, _L)])
                    parts = [acc[c] for c in range(_L)]
                    while len(parts) > 1:
                        parts = [a + b for a, b in zip(parts[::2], parts[1::2])]
                    rv = jnp.full((_L,), parts[0], jnp.float32)
                    onehot = jnp.maximum(
                        1.0 - jnp.abs(fiota - float(jj)), 0.0)
                    r_acc = r_acc + rv * onehot
                resbuf[pl.ds(e0, _L)] = r_acc
                return 0
            lax.fori_loop(0, _CHUNK // _L, group_body, 0)

        issue_gather(0, 0)
        issue_gather(1, 1)
        wait_gather(0, 0)
        compute(0, 0)
        issue_store(0, 0)

        def loop_body(i, _):
            for b in range(2):
                t = 1 + 2 * i + b
                p = (1 + b) % 2

                @pl.when(t + 1 < n_chunks)
                def _():
                    issue_gather(t + 1, 1 - p)
                wait_gather(t, p)
                compute(t, p)
                wait_store(t - 1, 1 - p)
                issue_store(t, p)
            return 0

        lax.fori_loop(0, (n_chunks - 1) // 2, loop_body, 0)
        wait_store(n_chunks - 1, 0)

    return k


def _pad1d(a, n):
    return jnp.concatenate([a, jnp.zeros((n - a.shape[0],), a.dtype)])


def _padded_len(e, gran):
    k = (e + gran - 1) // gran
    if k % 2 == 0:
        k += 1
    return k * gran


def kernel(x, edge_index, edge_label_index, Wl1, bl1, Wr1, br1, att1, bias1,
           Wl2, bl2, Wr2, br2, att2, bias2, Wp, bp):
    N = x.shape[0]
    E = edge_index.shape[1]
    E_tot = E + N
    E_pad1 = _padded_len(E_tot, _NW * 64)
    E_pad2 = _padded_len(E_tot, _NW * 128)
    EL = edge_label_index.shape[1]
    EL_pad = _padded_len(EL, _NW * _CHUNK)

    loop = jnp.arange(N, dtype=edge_index.dtype)
    src_raw = jnp.concatenate([edge_index[0], loop])
    dst_raw = jnp.concatenate([edge_index[1], loop])
    src1, dst1 = _pad1d(src_raw, E_pad1), _pad1d(dst_raw, E_pad1)
    src2, dst2 = _pad1d(src_raw, E_pad2), _pad1d(dst_raw, E_pad2)

    xl1, xr1 = _mm2(x, Wl1, bl1, Wr1, br1)
    acc1, den1 = _edge_sc(128, N, E_tot, E_pad1, 64)(xl1, xr1, src1, dst1, att1)
    xl2, xr2 = _norm_mm2(acc1, den1[:, 0, :].T, bias1, Wl2, bl2, Wr2, br2)

    acc2, den2 = _edge_sc(64, N, E_tot, E_pad2, 128)(xl2, xr2, src2, dst2, att2)
    wpv = Wp @ jnp.ones((2,), jnp.float32)
    h2, g2 = _norm_final(acc2, den2[:, 0, :].T, bias2, wpv)

    srcL = _pad1d(edge_label_index[0], EL_pad).reshape(-1, _CHUNK)
    dstL = _pad1d(edge_label_index[1], EL_pad).reshape(-1, _CHUNK)
    bps = jnp.full((_L,), jnp.sum(bp), jnp.float32)
    res = _decode_sc(64, N, EL, EL_pad)(g2, h2, srcL, dstL, bps)
    return res[:EL]
